# baseline (speedup 1.0000x reference)
import jax
import jax.numpy as jnp
from jax.experimental import pallas as pl

_THRESH = 0.6


def _h1_body(feat_ref, w1_ref, h1_ref):
    h1_ref[...] = jnp.dot(feat_ref[...], w1_ref[...],
                          preferred_element_type=jnp.float32)


def _h_body(adj_ref, h1_ref, w2_ref, hn_ref, y_ref):
    h = jnp.dot(adj_ref[...], h1_ref[...], preferred_element_type=jnp.float32)
    h = jnp.maximum(h, 0.0)
    norm = jnp.maximum(jnp.sqrt(jnp.sum(h * h, axis=1, keepdims=True)), 1e-12)
    hn_ref[...] = (h / norm).astype(jnp.bfloat16)
    y_ref[...] = jnp.dot(h, w2_ref[...],
                         preferred_element_type=jnp.float32).astype(jnp.bfloat16)


def _agg_body(hni_ref, hn_ref, y_ref, out_ref):
    sim = jax.lax.dot_general(hni_ref[...], hn_ref[...],
                              (((1,), (1,)), ((), ())),
                              preferred_element_type=jnp.float32)
    da = jnp.where(sim < _THRESH, 0.0, sim)
    l1 = jnp.maximum(jnp.sum(da, axis=1, keepdims=True), 1e-12)
    acc = jax.lax.dot_general(da.astype(jnp.bfloat16), y_ref[...],
                              (((1,), (0,)), ((), ())),
                              preferred_element_type=jnp.float32)
    out_ref[...] = acc / l1


def _pick_block(n, candidates):
    for c in candidates:
        if n % c == 0:
            return c
    return n


def kernel(feat, adj_spatial, W1, W2):
    n, in_feat = feat.shape
    hid = W1.shape[1]
    out_feat = W2.shape[1]
    f32 = jnp.float32

    b1 = _pick_block(n, [2000, 1000, 400, 200, 8])
    h1 = pl.pallas_call(
        _h1_body,
        grid=(n // b1,),
        in_specs=[
            pl.BlockSpec((b1, in_feat), lambda i: (i, 0)),
            pl.BlockSpec((in_feat, hid), lambda i: (0, 0)),
        ],
        out_specs=pl.BlockSpec((b1, hid), lambda i: (i, 0)),
        out_shape=jax.ShapeDtypeStruct((n, hid), f32),
    )(feat, W1)

    b2 = _pick_block(n, [400, 200, 8])
    hn, y = pl.pallas_call(
        _h_body,
        grid=(n // b2,),
        in_specs=[
            pl.BlockSpec((b2, n), lambda i: (i, 0)),
            pl.BlockSpec((n, hid), lambda i: (0, 0)),
            pl.BlockSpec((hid, out_feat), lambda i: (0, 0)),
        ],
        out_specs=[
            pl.BlockSpec((b2, hid), lambda i: (i, 0)),
            pl.BlockSpec((b2, out_feat), lambda i: (i, 0)),
        ],
        out_shape=[
            jax.ShapeDtypeStruct((n, hid), jnp.bfloat16),
            jax.ShapeDtypeStruct((n, out_feat), jnp.bfloat16),
        ],
    )(adj_spatial, h1, W2)

    b3 = _pick_block(n, [200, 8])
    out = pl.pallas_call(
        _agg_body,
        grid=(n // b3,),
        in_specs=[
            pl.BlockSpec((b3, hid), lambda i: (i, 0)),
            pl.BlockSpec((n, hid), lambda i: (0, 0)),
            pl.BlockSpec((n, out_feat), lambda i: (0, 0)),
        ],
        out_specs=pl.BlockSpec((b3, out_feat), lambda i: (i, 0)),
        out_shape=jax.ShapeDtypeStruct((n, out_feat), f32),
    )(hn, hn, y)

    return out


# speedup vs baseline: 1.0621x; 1.0621x over previous
import jax
import jax.numpy as jnp
from jax.experimental import pallas as pl

_THRESH = 0.6


def _h1_body(feat_ref, w1_ref, h1_ref):
    h1_ref[...] = jnp.dot(feat_ref[...], w1_ref[...],
                          preferred_element_type=jnp.float32)


def _h_body(adj_ref, h1_ref, w2_ref, hn_ref, y_ref):
    h = jnp.dot(adj_ref[...], h1_ref[...], preferred_element_type=jnp.float32)
    h = jnp.maximum(h, 0.0)
    norm = jnp.maximum(jnp.sqrt(jnp.sum(h * h, axis=1, keepdims=True)), 1e-12)
    hn_ref[...] = (h / norm).astype(jnp.bfloat16)
    y_ref[...] = jnp.dot(h, w2_ref[...],
                         preferred_element_type=jnp.float32).astype(jnp.bfloat16)


def _agg_body(hni_ref, hn_ref, y_ref, out_ref):
    sim = jax.lax.dot_general(hni_ref[...], hn_ref[...],
                              (((1,), (1,)), ((), ())),
                              preferred_element_type=jnp.float32)
    da = jnp.where(sim < _THRESH, 0.0, sim)
    l1 = jnp.maximum(jnp.sum(da, axis=1, keepdims=True), 1e-12)
    acc = jax.lax.dot_general(da.astype(jnp.bfloat16), y_ref[...],
                              (((1,), (0,)), ((), ())),
                              preferred_element_type=jnp.float32)
    out_ref[...] = acc / l1


def _pick_block(n, candidates):
    for c in candidates:
        if n % c == 0:
            return c
    return n


def kernel(feat, adj_spatial, W1, W2):
    n, in_feat = feat.shape
    hid = W1.shape[1]
    out_feat = W2.shape[1]
    f32 = jnp.float32

    b1 = _pick_block(n, [2000, 1000, 400, 200, 8])
    h1 = pl.pallas_call(
        _h1_body,
        grid=(n // b1,),
        in_specs=[
            pl.BlockSpec((b1, in_feat), lambda i: (i, 0)),
            pl.BlockSpec((in_feat, hid), lambda i: (0, 0)),
        ],
        out_specs=pl.BlockSpec((b1, hid), lambda i: (i, 0)),
        out_shape=jax.ShapeDtypeStruct((n, hid), f32),
    )(feat, W1)

    b2 = _pick_block(n, [400, 200, 8])
    hn, y = pl.pallas_call(
        _h_body,
        grid=(n // b2,),
        in_specs=[
            pl.BlockSpec((b2, n), lambda i: (i, 0)),
            pl.BlockSpec((n, hid), lambda i: (0, 0)),
            pl.BlockSpec((hid, out_feat), lambda i: (0, 0)),
        ],
        out_specs=[
            pl.BlockSpec((b2, hid), lambda i: (i, 0)),
            pl.BlockSpec((b2, out_feat), lambda i: (i, 0)),
        ],
        out_shape=[
            jax.ShapeDtypeStruct((n, hid), jnp.bfloat16),
            jax.ShapeDtypeStruct((n, out_feat), jnp.bfloat16),
        ],
    )(adj_spatial, h1, W2)

    b3 = _pick_block(n, [1000, 400, 200, 8])
    out = pl.pallas_call(
        _agg_body,
        grid=(n // b3,),
        in_specs=[
            pl.BlockSpec((b3, hid), lambda i: (i, 0)),
            pl.BlockSpec((n, hid), lambda i: (0, 0)),
            pl.BlockSpec((n, out_feat), lambda i: (0, 0)),
        ],
        out_specs=pl.BlockSpec((b3, out_feat), lambda i: (i, 0)),
        out_shape=jax.ShapeDtypeStruct((n, out_feat), f32),
    )(hn, hn, y)

    return out
